# Initial kernel scaffold; baseline (speedup 1.0000x reference)
import functools

import jax
import jax.numpy as jnp
from jax import lax
from jax.experimental import pallas as pl
from jax.experimental.pallas import tpu as pltpu
from jax.experimental.pallas import tpu_sc as plsc

N = 10000
E = 160000
D_NODE = 128
D_EDGE = 16
H = 200
HP = 208
G = 64

NW = 32
C = 128
NCHUNK = 40
EPT = NCHUNK * C
EPAD = NW * EPT
NPT = N // 16

BN = 1000
BE = 1024

_f32 = jnp.float32


def _relu(x):
    return jnp.maximum(x, 0.0)


def _sigmoid(x):
    return 1.0 / (1.0 + jnp.exp(-x))


def _node_prep_body(x_ref, npW_ref, npb_ref, mWh_ref, aWs_ref, aWd_ref,
                    P_ref, as_ref, ad_ref):
    h = _relu(jnp.dot(x_ref[...], npW_ref[...],
                      preferred_element_type=_f32) + npb_ref[...])
    P_ref[...] = jnp.dot(h, mWh_ref[...], preferred_element_type=_f32)
    as_ref[...] = jnp.dot(h, aWs_ref[...], preferred_element_type=_f32)
    ad_ref[...] = jnp.dot(h, aWd_ref[...], preferred_element_type=_f32)


def _node_prep(x, npW, npb, mWh, aWs, aWd):
    nb = N // BN
    return pl.pallas_call(
        _node_prep_body,
        grid=(nb,),
        in_specs=[
            pl.BlockSpec((BN, D_NODE), lambda i: (i, 0)),
            pl.BlockSpec((D_NODE, HP), lambda i: (0, 0)),
            pl.BlockSpec((1, HP), lambda i: (0, 0)),
            pl.BlockSpec((HP, HP), lambda i: (0, 0)),
            pl.BlockSpec((HP, 1), lambda i: (0, 0)),
            pl.BlockSpec((HP, 1), lambda i: (0, 0)),
        ],
        out_specs=[
            pl.BlockSpec((BN, HP), lambda i: (i, 0)),
            pl.BlockSpec((BN, 1), lambda i: (i, 0)),
            pl.BlockSpec((BN, 1), lambda i: (i, 0)),
        ],
        out_shape=[
            jax.ShapeDtypeStruct((N, HP), _f32),
            jax.ShapeDtypeStruct((N, 1), _f32),
            jax.ShapeDtypeStruct((N, 1), _f32),
        ],
    )(x, npW, npb, mWh, aWs, aWd)


def _edge_prep_body(ef_ref, epW_ref, epb_ref,
                    mWe0_ref, mb0_ref, aWe0_ref, ab0_ref,
                    mWe1_ref, mb1_ref, aWe1_ref, ab1_ref,
                    Q0_ref, a0_ref, Q1_ref, a1_ref):
    e = _relu(jnp.dot(ef_ref[...], epW_ref[...],
                      preferred_element_type=_f32) + epb_ref[...])
    Q0_ref[...] = jnp.dot(e, mWe0_ref[...], preferred_element_type=_f32) + mb0_ref[...]
    Q1_ref[...] = jnp.dot(e, mWe1_ref[...], preferred_element_type=_f32) + mb1_ref[...]
    eid = pl.program_id(0) * BE + lax.broadcasted_iota(jnp.int32, (BE, 1), 0)
    valid = eid < E
    a0 = jnp.dot(e, aWe0_ref[...], preferred_element_type=_f32) + ab0_ref[...]
    a1 = jnp.dot(e, aWe1_ref[...], preferred_element_type=_f32) + ab1_ref[...]
    a0_ref[...] = jnp.where(valid, a0, -1e4)
    a1_ref[...] = jnp.where(valid, a1, -1e4)


def _edge_prep(ef, epW, epb, mWe0, mb0, aWe0, ab0, mWe1, mb1, aWe1, ab1):
    nb = EPAD // BE
    w_spec = lambda shape: pl.BlockSpec(shape, lambda i: (0, 0))
    return pl.pallas_call(
        _edge_prep_body,
        grid=(nb,),
        in_specs=[
            pl.BlockSpec((BE, D_EDGE), lambda i: (i, 0)),
            w_spec((D_EDGE, HP)), w_spec((1, HP)),
            w_spec((HP, HP)), w_spec((1, HP)), w_spec((HP, 1)), w_spec((1, 1)),
            w_spec((HP, HP)), w_spec((1, HP)), w_spec((HP, 1)), w_spec((1, 1)),
        ],
        out_specs=[
            pl.BlockSpec((BE, HP), lambda i: (i, 0)),
            pl.BlockSpec((BE, 1), lambda i: (i, 0)),
            pl.BlockSpec((BE, HP), lambda i: (i, 0)),
            pl.BlockSpec((BE, 1), lambda i: (i, 0)),
        ],
        out_shape=[
            jax.ShapeDtypeStruct((EPAD, HP), _f32),
            jax.ShapeDtypeStruct((EPAD, 1), _f32),
            jax.ShapeDtypeStruct((EPAD, HP), _f32),
            jax.ShapeDtypeStruct((EPAD, 1), _f32),
        ],
    )(ef, epW, epb, mWe0, mb0, aWe0, ab0, mWe1, mb1, aWe1, ab1)


@functools.partial(
    pl.kernel,
    out_type=jax.ShapeDtypeStruct((2, N, HP), _f32),
    mesh=plsc.VectorSubcoreMesh(core_axis_name="c", subcore_axis_name="s"),
    scratch_types=[
        pltpu.VMEM_SHARED((N, HP), _f32),
        pltpu.VMEM((N,), _f32),
        pltpu.VMEM((N,), _f32),
        pltpu.VMEM((C,), jnp.int32),
        pltpu.VMEM((C,), jnp.int32),
        pltpu.VMEM((C,), _f32),
        pltpu.VMEM((C,), _f32),
        pltpu.VMEM((C, HP), _f32),
        pltpu.VMEM((C, HP), _f32),
        pltpu.VMEM((C, HP), _f32),
        pltpu.SemaphoreType.DMA,
    ],
)
def _sc_edge(P_hbm, Q_hbm, as_hbm, ad_hbm, ae_hbm, src_hbm, dst_hbm,
             out_hbm,
             acc, as_v, ad_v, src_v, dst_v, ae_v, alpha_v,
             prow_v, qrow_v, msg_v, gsem):
    c = lax.axis_index("c")
    s = lax.axis_index("s")
    wid = c * 16 + s

    def _zrow(j, carry):
        for k in range(HP // 16):
            msg_v[j, pl.ds(k * 16, 16)] = jnp.zeros((16,), _f32)
        return carry
    lax.fori_loop(0, 125, _zrow, 0)

    def _zcp(i, carry):
        pltpu.sync_copy(msg_v.at[pl.ds(0, 125)],
                        acc.at[pl.ds(s * NPT + i * 125, 125)])
        return carry
    lax.fori_loop(0, NPT // 125, _zcp, 0)

    pltpu.sync_copy(as_hbm, as_v)
    pltpu.sync_copy(ad_hbm, ad_v)
    plsc.subcore_barrier()

    base = wid * EPT

    def _chunk(i, carry):
        eb = base + i * C
        pltpu.sync_copy(src_hbm.at[pl.ds(eb, C)], src_v)
        pltpu.sync_copy(dst_hbm.at[pl.ds(eb, C)], dst_v)
        pltpu.sync_copy(ae_hbm.at[pl.ds(eb, C)], ae_v)
        pltpu.sync_copy(Q_hbm.at[pl.ds(eb, C)], qrow_v)
        pltpu.async_copy(P_hbm.at[src_v], prow_v, gsem).wait()
        for j in range(C // 16):
            sv = src_v[pl.ds(j * 16, 16)]
            dv = dst_v[pl.ds(j * 16, 16)]
            x = (plsc.load_gather(as_v, [sv])
                 + plsc.load_gather(ad_v, [dv])
                 + ae_v[pl.ds(j * 16, 16)])
            alpha_v[pl.ds(j * 16, 16)] = _sigmoid(x)

        def _edge(j, carry2):
            a = alpha_v[j]
            for k in range(HP // 16):
                p = prow_v[j, pl.ds(k * 16, 16)]
                q = qrow_v[j, pl.ds(k * 16, 16)]
                msg_v[j, pl.ds(k * 16, 16)] = _relu(p + q) * a
            return carry2
        lax.fori_loop(0, C, _edge, 0)
        pltpu.sync_copy(msg_v, acc.at[dst_v], add=True)
        return carry

    lax.fori_loop(0, NCHUNK, _chunk, 0)
    plsc.subcore_barrier()
    pltpu.sync_copy(acc.at[pl.ds(s * NPT, NPT)],
                    out_hbm.at[c, pl.ds(s * NPT, NPT)])


def _combine_body(part_ref, g_ref, b_ref, mWh_ref, aWs_ref, aWd_ref,
                  h_ref, P_ref, as_ref, ad_ref):
    agg = part_ref[0] + part_ref[1]
    mu = jnp.sum(agg, axis=1, keepdims=True) / H
    var = jnp.sum(agg * agg, axis=1, keepdims=True) / H - mu * mu
    hn = _relu((agg - mu) * lax.rsqrt(var + 1e-5) * g_ref[...] + b_ref[...])
    h_ref[...] = hn
    P_ref[...] = jnp.dot(hn, mWh_ref[...], preferred_element_type=_f32)
    as_ref[...] = jnp.dot(hn, aWs_ref[...], preferred_element_type=_f32)
    ad_ref[...] = jnp.dot(hn, aWd_ref[...], preferred_element_type=_f32)


def _combine(parts, g, b, mWh, aWs, aWd):
    nb = N // BN
    return pl.pallas_call(
        _combine_body,
        grid=(nb,),
        in_specs=[
            pl.BlockSpec((2, BN, HP), lambda i: (0, i, 0)),
            pl.BlockSpec((1, HP), lambda i: (0, 0)),
            pl.BlockSpec((1, HP), lambda i: (0, 0)),
            pl.BlockSpec((HP, HP), lambda i: (0, 0)),
            pl.BlockSpec((HP, 1), lambda i: (0, 0)),
            pl.BlockSpec((HP, 1), lambda i: (0, 0)),
        ],
        out_specs=[
            pl.BlockSpec((BN, HP), lambda i: (i, 0)),
            pl.BlockSpec((BN, HP), lambda i: (i, 0)),
            pl.BlockSpec((BN, 1), lambda i: (i, 0)),
            pl.BlockSpec((BN, 1), lambda i: (i, 0)),
        ],
        out_shape=[
            jax.ShapeDtypeStruct((N, HP), _f32),
            jax.ShapeDtypeStruct((N, HP), _f32),
            jax.ShapeDtypeStruct((N, 1), _f32),
            jax.ShapeDtypeStruct((N, 1), _f32),
        ],
    )(parts, g, b, mWh, aWs, aWd)


def _readout_body(h_ref, bcol_ref, brow_ref, roW_ref, rob_ref,
                  WihR_ref, WihZ_ref, WihN_ref, bihR_ref, bihZ_ref, bihN_ref,
                  WhhR_ref, WhhZ_ref, WhhN_ref, bhhR_ref, bhhZ_ref, bhhN_ref,
                  outW_ref, outb_ref, out_ref):
    h = h_ref[...]
    Bm = jnp.where(bcol_ref[...] == lax.broadcasted_iota(jnp.int32, (N, G), 1),
                   1.0, 0.0)
    BmT = jnp.where(brow_ref[...] == lax.broadcasted_iota(jnp.int32, (G, N), 0),
                    1.0, 0.0)
    counts = jnp.maximum(jnp.sum(BmT, axis=1, keepdims=True), 1.0)
    gh = jnp.dot(BmT, h, preferred_element_type=_f32) / counts
    for _ in range(2):
        ctx = jnp.dot(Bm, gh, preferred_element_type=_f32)
        ap = _sigmoid(jnp.dot(h * ctx, roW_ref[...],
                              preferred_element_type=_f32) + rob_ref[...])
        context = jnp.dot(BmT, ap * h, preferred_element_type=_f32)
        i_r = jnp.dot(context, WihR_ref[...], preferred_element_type=_f32) + bihR_ref[...]
        i_z = jnp.dot(context, WihZ_ref[...], preferred_element_type=_f32) + bihZ_ref[...]
        i_n = jnp.dot(context, WihN_ref[...], preferred_element_type=_f32) + bihN_ref[...]
        h_r = jnp.dot(gh, WhhR_ref[...], preferred_element_type=_f32) + bhhR_ref[...]
        h_z = jnp.dot(gh, WhhZ_ref[...], preferred_element_type=_f32) + bhhZ_ref[...]
        h_n = jnp.dot(gh, WhhN_ref[...], preferred_element_type=_f32) + bhhN_ref[...]
        r = _sigmoid(i_r + h_r)
        z = _sigmoid(i_z + h_z)
        n = jnp.tanh(i_n + r * h_n)
        gh = (1.0 - z) * n + z * gh
    out_ref[...] = jnp.dot(gh, outW_ref[...],
                           preferred_element_type=_f32) + outb_ref[...]


def _readout(h, bcol, brow, roW, rob, Wih, bih, Whh, bhh, outW, outb):
    return pl.pallas_call(
        _readout_body,
        out_shape=jax.ShapeDtypeStruct((G, 1), _f32),
    )(h, bcol, brow, roW, rob, *Wih, *bih, *Whh, *bhh, outW, outb)


def _padc(w, cols=HP):
    return jnp.pad(w, ((0, 0), (0, cols - w.shape[1])))


def _padrc(w):
    return jnp.pad(w, ((0, HP - w.shape[0]), (0, HP - w.shape[1])))


def _padr(w):
    return jnp.pad(w, ((0, HP - w.shape[0]), (0, 0)))


def _padv(v):
    return jnp.pad(v, (0, HP - v.shape[0])).reshape(1, HP)


def kernel(node_features, edge_features, edge_index, batch,
           np_W, np_b, ep_W, ep_b,
           l0_attn_W, l0_attn_b, l0_msg_W, l0_msg_b, l0_ln_g, l0_ln_b,
           l1_attn_W, l1_attn_b, l1_msg_W, l1_msg_b, l1_ln_g, l1_ln_b,
           ro_W, ro_b, gru_Wih, gru_bih, gru_Whh, gru_bhh, out_W, out_b):
    npW = _padc(np_W); npb = _padv(np_b)
    epW = _padc(ep_W); epb = _padv(ep_b)
    aW0s = _padr(l0_attn_W[:H]); aW0d = _padr(l0_attn_W[H:2 * H])
    aW0e = _padr(l0_attn_W[2 * H:]); ab0 = l0_attn_b.reshape(1, 1)
    aW1s = _padr(l1_attn_W[:H]); aW1d = _padr(l1_attn_W[H:2 * H])
    aW1e = _padr(l1_attn_W[2 * H:]); ab1 = l1_attn_b.reshape(1, 1)
    mW0h = _padrc(l0_msg_W[:H]); mW0e = _padrc(l0_msg_W[H:])
    mW1h = _padrc(l1_msg_W[:H]); mW1e = _padrc(l1_msg_W[H:])
    mb0 = _padv(l0_msg_b); mb1 = _padv(l1_msg_b)
    g0 = _padv(l0_ln_g); b0 = _padv(l0_ln_b)
    g1 = _padv(l1_ln_g); b1 = _padv(l1_ln_b)
    roW = _padr(ro_W); rob = ro_b.reshape(1, 1)
    Wih = [_padrc(gru_Wih[:, i * H:(i + 1) * H]) for i in range(3)]
    bih = [_padv(gru_bih[i * H:(i + 1) * H]) for i in range(3)]
    Whh = [_padrc(gru_Whh[:, i * H:(i + 1) * H]) for i in range(3)]
    bhh = [_padv(gru_bhh[i * H:(i + 1) * H]) for i in range(3)]
    outW = _padr(out_W); outb = out_b.reshape(1, 1)

    src = jnp.pad(edge_index[0], (0, EPAD - E))
    dst = jnp.pad(edge_index[1], (0, EPAD - E))
    efp = jnp.pad(edge_features, ((0, EPAD - E), (0, 0)))
    bcol = batch.reshape(N, 1)
    brow = batch.reshape(1, N)

    P0, a0s, a0d = _node_prep(node_features, npW, npb, mW0h, aW0s, aW0d)
    Q0, a0e, Q1, a1e = _edge_prep(efp, epW, epb, mW0e, mb0, aW0e, ab0,
                                  mW1e, mb1, aW1e, ab1)

    parts0 = _sc_edge(P0, Q0, a0s.reshape(N), a0d.reshape(N),
                      a0e.reshape(EPAD), src, dst)
    _, P1, a1s, a1d = _combine(parts0, g0, b0, mW1h, aW1s, aW1d)

    parts1 = _sc_edge(P1, Q1, a1s.reshape(N), a1d.reshape(N),
                      a1e.reshape(EPAD), src, dst)
    h2, _, _, _ = _combine(parts1, g1, b1, mW1h, aW1s, aW1d)

    return _readout(h2, bcol, brow, roW, rob, Wih, bih, Whh, bhh, outW, outb)


# speedup vs baseline: 1.1372x; 1.1372x over previous
import functools

import jax
import jax.numpy as jnp
from jax import lax
from jax.experimental import pallas as pl
from jax.experimental.pallas import tpu as pltpu
from jax.experimental.pallas import tpu_sc as plsc

N = 10000
E = 160000
D_NODE = 128
D_EDGE = 16
H = 200
HP = 256
HH = 128
PW = 144
G = 64

NT = 16
C = 96
NCHUNK = 105
EPT = NCHUNK * C
EPAD = NT * EPT
C2 = 112
EPT2 = EPAD // 32
NCHUNK2 = EPT2 // C2

BN = 1000
BE = 1024
NEB = EPAD // BE

_f32 = jnp.float32
_HI = lax.Precision.HIGHEST
_bf16 = jnp.bfloat16


def _bdot(a, b):
    return jnp.dot(a.astype(_bf16), b.astype(_bf16),
                   preferred_element_type=_f32)


def _relu(x):
    return jnp.maximum(x, 0.0)


def _sigmoid(x):
    return 1.0 / (1.0 + jnp.exp(-x))


def _node_prep_body(x_ref, npW_ref, npb_ref, mWa_ref, mWb_ref, aWs_ref,
                    aWd_ref, Pa_ref, Pb_ref, D_ref, S_ref):
    h = _relu(jnp.dot(x_ref[...], npW_ref[...],
                      preferred_element_type=_f32, precision=_HI) + npb_ref[...])
    asv = jnp.dot(h, aWs_ref[...], preferred_element_type=_f32, precision=_HI)
    adv = jnp.dot(h, aWd_ref[...], preferred_element_type=_f32, precision=_HI)
    z15 = jnp.zeros((h.shape[0], 15), _f32)
    Pa_ref[...] = jnp.concatenate(
        [jnp.dot(h, mWa_ref[...], preferred_element_type=_f32, precision=_HI),
         asv, z15], 1)
    Pb_ref[...] = jnp.concatenate(
        [jnp.dot(h, mWb_ref[...], preferred_element_type=_f32, precision=_HI),
         asv, z15], 1)
    D_ref[...] = jnp.concatenate([adv, z15], 1)
    S_ref[...] = jnp.concatenate([asv, z15], 1)


def _node_prep(x, npW, npb, mWa, mWb, aWs, aWd):
    nb = N // BN
    w = lambda shape: pl.BlockSpec(shape, lambda i: (0, 0))
    return pl.pallas_call(
        _node_prep_body,
        grid=(nb,),
        in_specs=[
            pl.BlockSpec((BN, D_NODE), lambda i: (i, 0)),
            w((D_NODE, HP)), w((1, HP)),
            w((HP, HH)), w((HP, HH)), w((HP, 1)), w((HP, 1)),
        ],
        out_specs=[
            pl.BlockSpec((BN, PW), lambda i: (i, 0)),
            pl.BlockSpec((BN, PW), lambda i: (i, 0)),
            pl.BlockSpec((BN, 16), lambda i: (i, 0)),
            pl.BlockSpec((BN, 16), lambda i: (i, 0)),
        ],
        out_shape=[
            jax.ShapeDtypeStruct((N, PW), _f32),
            jax.ShapeDtypeStruct((N, PW), _f32),
            jax.ShapeDtypeStruct((N, 16), _f32),
            jax.ShapeDtypeStruct((N, 16), _f32),
        ],
    )(x, npW, npb, mWa, mWb, aWs, aWd)


def _edge_prep_body(ef_ref, epW_ref, epb_ref,
                    mWea0_ref, mWeb0_ref, mb0_ref, aWe0_ref, ab0_ref,
                    mWea1_ref, mWeb1_ref, mb1_ref, aWe1_ref, ab1_ref,
                    Qa0_ref, Qb0_ref, a0_ref, Qa1_ref, Qb1_ref, a1_ref):
    e = _relu(jnp.dot(ef_ref[...], epW_ref[...],
                      preferred_element_type=_f32, precision=_HI) + epb_ref[...])
    Qa0_ref[...] = _bdot(e, mWea0_ref[...]) + mb0_ref[:, :HH]
    Qb0_ref[...] = _bdot(e, mWeb0_ref[...]) + mb0_ref[:, HH:]
    Qa1_ref[...] = _bdot(e, mWea1_ref[...]) + mb1_ref[:, :HH]
    Qb1_ref[...] = _bdot(e, mWeb1_ref[...]) + mb1_ref[:, HH:]
    eid = pl.program_id(0) * BE + lax.broadcasted_iota(jnp.int32, (BE, 1), 0)
    valid = eid < E
    a0 = _bdot(e, aWe0_ref[...]) + ab0_ref[...]
    a1 = _bdot(e, aWe1_ref[...]) + ab1_ref[...]
    a0_ref[...] = jnp.where(valid, a0, -1e4)
    a1_ref[...] = jnp.where(valid, a1, -1e4)


def _edge_prep(ef, epW, epb, mWea0, mWeb0, mb0, aWe0, ab0,
               mWea1, mWeb1, mb1, aWe1, ab1):
    epad_blk = -(-EPAD // BE) * BE
    nb = epad_blk // BE
    w = lambda shape: pl.BlockSpec(shape, lambda i: (0, 0))
    outs = pl.pallas_call(
        _edge_prep_body,
        grid=(nb,),
        in_specs=[
            pl.BlockSpec((BE, D_EDGE), lambda i: (i, 0)),
            w((D_EDGE, HP)), w((1, HP)),
            w((HP, HH)), w((HP, HH)), w((1, HP)), w((HP, 1)), w((1, 1)),
            w((HP, HH)), w((HP, HH)), w((1, HP)), w((HP, 1)), w((1, 1)),
        ],
        out_specs=[
            pl.BlockSpec((BE, HH), lambda i: (i, 0)),
            pl.BlockSpec((BE, HH), lambda i: (i, 0)),
            pl.BlockSpec((BE, 1), lambda i: (i, 0)),
            pl.BlockSpec((BE, HH), lambda i: (i, 0)),
            pl.BlockSpec((BE, HH), lambda i: (i, 0)),
            pl.BlockSpec((BE, 1), lambda i: (i, 0)),
        ],
        out_shape=[
            jax.ShapeDtypeStruct((epad_blk, HH), _f32),
            jax.ShapeDtypeStruct((epad_blk, HH), _f32),
            jax.ShapeDtypeStruct((epad_blk, 1), _f32),
            jax.ShapeDtypeStruct((epad_blk, HH), _f32),
            jax.ShapeDtypeStruct((epad_blk, HH), _f32),
            jax.ShapeDtypeStruct((epad_blk, 1), _f32),
        ],
    )(ef, epW, epb, mWea0, mWeb0, mb0, aWe0, ab0,
      mWea1, mWeb1, mb1, aWe1, ab1)
    return [o[:EPAD] for o in outs]


def _sc_edge_body(Pa_hbm, Pb_hbm, Qa_hbm, Qb_hbm, al_hbm,
                  src_hbm, dst_hbm, out_hbm,
                  acc, src_v, dst_v, al_v, prow_v, qrow_v, msg_v,
                  gsem):
    c = lax.axis_index("c")
    s = lax.axis_index("s")

    def _zrow(j, carry):
        for k in range(HH // 16):
            msg_v[j, pl.ds(k * 16, 16)] = jnp.zeros((16,), _f32)
        return carry
    lax.fori_loop(0, C, _zrow, 0)

    def _zcp(i, carry):
        pltpu.sync_copy(msg_v.at[pl.ds(0, 16)],
                        acc.at[pl.ds(s * 624 + i * 16, 16)])
        return carry
    lax.fori_loop(0, 39, _zcp, 0)

    @pl.when(s == 15)
    def _ztail():
        pltpu.sync_copy(msg_v.at[pl.ds(0, 16)], acc.at[pl.ds(9984, 16)])

    plsc.subcore_barrier()

    base = s * EPT

    def _chunk(i, carry):
        eb = base + i * C
        pltpu.sync_copy(src_hbm.at[pl.ds(eb, C)], src_v)
        pltpu.sync_copy(dst_hbm.at[pl.ds(eb, C)], dst_v)
        pltpu.sync_copy(al_hbm.at[pl.ds(eb, C)], al_v.at[pl.ds(0, C)])

        @pl.when(c == 0)
        def _half_a():
            pltpu.sync_copy(Qa_hbm.at[pl.ds(eb, C)], qrow_v)
            pltpu.async_copy(Pa_hbm.at[src_v], prow_v, gsem).wait()

        @pl.when(c == 1)
        def _half_b():
            pltpu.sync_copy(Qb_hbm.at[pl.ds(eb, C)], qrow_v)
            pltpu.async_copy(Pb_hbm.at[src_v], prow_v, gsem).wait()

        def _edge(j, carry2):
            a = al_v[pl.ds(j, 16)][0]
            for k in range(HH // 16):
                p = prow_v[j, pl.ds(k * 16, 16)]
                q = qrow_v[j, pl.ds(k * 16, 16)]
                msg_v[j, pl.ds(k * 16, 16)] = _relu(p + q) * a
            return carry2
        lax.fori_loop(0, C, _edge, 0)

        pltpu.sync_copy(msg_v, acc.at[dst_v], add=True)
        return carry

    lax.fori_loop(0, NCHUNK, _chunk, 0)
    plsc.subcore_barrier()

    pltpu.sync_copy(acc.at[pl.ds(s * 624, 624)],
                    out_hbm.at[c, pl.ds(s * 624, 624)])

    @pl.when(s == 15)
    def _wtail():
        pltpu.sync_copy(acc.at[pl.ds(9984, 16)],
                        out_hbm.at[c, pl.ds(9984, 16)])


@functools.cache
def _get_sc_edge():
    return pl.kernel(
        _sc_edge_body,
        out_type=jax.ShapeDtypeStruct((2, N, HH), _f32),
        mesh=plsc.VectorSubcoreMesh(core_axis_name="c", subcore_axis_name="s"),
        compiler_params=pltpu.CompilerParams(needs_layout_passes=False,
                                             use_tc_tiling_on_sc=False),
        scratch_types=[
            pltpu.VMEM_SHARED((N, HH), _f32),
            pltpu.VMEM((C,), jnp.int32),
            pltpu.VMEM((C,), jnp.int32),
            pltpu.VMEM((C + 16,), _f32),
            pltpu.VMEM((C, PW), _f32),
            pltpu.VMEM((C, HH), _f32),
            pltpu.VMEM((C, HH), _f32),
            pltpu.SemaphoreType.DMA,
        ],
    )


def _sc_logit_body(S_hbm, D_hbm, ae_hbm, src_hbm, dst_hbm, out_hbm,
                   src_v, dst_v, ae_v, srow_v, drow_v, lg_v, ssem, dsem):
    c = lax.axis_index("c")
    s = lax.axis_index("s")
    wid = c * 16 + s
    base = wid * EPT2
    lane0 = lax.iota(jnp.int32, 16) == 0

    def _chunk(i, carry):
        eb = base + i * C2
        pltpu.sync_copy(src_hbm.at[pl.ds(eb, C2)], src_v)
        pltpu.sync_copy(dst_hbm.at[pl.ds(eb, C2)], dst_v)
        pltpu.sync_copy(ae_hbm.at[pl.ds(eb, C2)], ae_v.at[pl.ds(0, C2)])
        pltpu.async_copy(S_hbm.at[src_v], srow_v, ssem).wait()
        pltpu.async_copy(D_hbm.at[dst_v], drow_v, dsem).wait()

        def _edge(j, carry2):
            lg = (srow_v[j, pl.ds(0, 16)] + drow_v[j, pl.ds(0, 16)]
                  + ae_v[pl.ds(j, 16)])
            plsc.store_compressed(lg_v.at[pl.ds(j, 16)], lg, mask=lane0)
            return carry2
        lax.fori_loop(0, C2, _edge, 0)
        pltpu.sync_copy(lg_v.at[pl.ds(0, C2)], out_hbm.at[pl.ds(eb, C2)])
        return carry

    lax.fori_loop(0, NCHUNK2, _chunk, 0)


@functools.cache
def _get_sc_logit():
    return pl.kernel(
        _sc_logit_body,
        out_type=jax.ShapeDtypeStruct((EPAD,), _f32),
        mesh=plsc.VectorSubcoreMesh(core_axis_name="c", subcore_axis_name="s"),
        compiler_params=pltpu.CompilerParams(needs_layout_passes=False,
                                             use_tc_tiling_on_sc=False),
        scratch_types=[
            pltpu.VMEM((C2,), jnp.int32),
            pltpu.VMEM((C2,), jnp.int32),
            pltpu.VMEM((C2 + 16,), _f32),
            pltpu.VMEM((C2, 16), _f32),
            pltpu.VMEM((C2, 16), _f32),
            pltpu.VMEM((C2 + 16,), _f32),
            pltpu.SemaphoreType.DMA,
            pltpu.SemaphoreType.DMA,
        ],
    )


def _sig_body(x_ref, o_ref):
    o_ref[...] = jax.nn.sigmoid(x_ref[...])


def _tc_sigmoid(logits):
    return pl.pallas_call(
        _sig_body,
        grid=(32,),
        in_specs=[pl.BlockSpec((EPT2, 1), lambda i: (i, 0))],
        out_specs=pl.BlockSpec((EPT2, 1), lambda i: (i, 0)),
        out_shape=jax.ShapeDtypeStruct((EPAD, 1), _f32),
    )(logits.reshape(EPAD, 1)).reshape(EPAD)


def _combine_body(part_ref, g_ref, b_ref, mWa_ref, mWb_ref, aWs_ref,
                  aWd_ref, h_ref, Pa_ref, Pb_ref, D_ref, S_ref):
    agg = jnp.concatenate([part_ref[0], part_ref[1]], 1)
    mu = jnp.sum(agg, axis=1, keepdims=True) / H
    var = jnp.sum(agg * agg, axis=1, keepdims=True) / H - mu * mu
    hn = _relu((agg - mu) / jnp.sqrt(var + 1e-5) * g_ref[...] + b_ref[...])
    h_ref[...] = hn
    asv = jnp.dot(hn, aWs_ref[...], preferred_element_type=_f32, precision=_HI)
    adv = jnp.dot(hn, aWd_ref[...], preferred_element_type=_f32, precision=_HI)
    z15 = jnp.zeros((hn.shape[0], 15), _f32)
    Pa_ref[...] = jnp.concatenate(
        [jnp.dot(hn, mWa_ref[...], preferred_element_type=_f32, precision=_HI),
         asv, z15], 1)
    Pb_ref[...] = jnp.concatenate(
        [jnp.dot(hn, mWb_ref[...], preferred_element_type=_f32, precision=_HI),
         asv, z15], 1)
    D_ref[...] = jnp.concatenate([adv, z15], 1)
    S_ref[...] = jnp.concatenate([asv, z15], 1)


def _combine(parts, g, b, mWa, mWb, aWs, aWd):
    nb = N // BN
    w = lambda shape: pl.BlockSpec(shape, lambda i: (0, 0))
    return pl.pallas_call(
        _combine_body,
        grid=(nb,),
        in_specs=[
            pl.BlockSpec((2, BN, HH), lambda i: (0, i, 0)),
            w((1, HP)), w((1, HP)),
            w((HP, HH)), w((HP, HH)), w((HP, 1)), w((HP, 1)),
        ],
        out_specs=[
            pl.BlockSpec((BN, HP), lambda i: (i, 0)),
            pl.BlockSpec((BN, PW), lambda i: (i, 0)),
            pl.BlockSpec((BN, PW), lambda i: (i, 0)),
            pl.BlockSpec((BN, 16), lambda i: (i, 0)),
            pl.BlockSpec((BN, 16), lambda i: (i, 0)),
        ],
        out_shape=[
            jax.ShapeDtypeStruct((N, HP), _f32),
            jax.ShapeDtypeStruct((N, PW), _f32),
            jax.ShapeDtypeStruct((N, PW), _f32),
            jax.ShapeDtypeStruct((N, 16), _f32),
            jax.ShapeDtypeStruct((N, 16), _f32),
        ],
    )(parts, g, b, mWa, mWb, aWs, aWd)


def _seg_mean_body(h_ref, brow_ref, gh_ref, cnt_ref):
    i = pl.program_id(0)

    @pl.when(i == 0)
    def _init():
        gh_ref[...] = jnp.zeros_like(gh_ref)
        cnt_ref[...] = jnp.zeros_like(cnt_ref)

    BmT = jnp.where(brow_ref[0] == lax.broadcasted_iota(jnp.int32, (G, BN), 0),
                    1.0, 0.0)
    gh_ref[...] += jnp.dot(BmT, h_ref[...],
                           preferred_element_type=_f32, precision=_HI)
    cnt_ref[...] += jnp.sum(BmT, axis=1, keepdims=True)


def _seg_mean(h, brow):
    nb = N // BN
    return pl.pallas_call(
        _seg_mean_body,
        grid=(nb,),
        in_specs=[
            pl.BlockSpec((BN, HP), lambda i: (i, 0)),
            pl.BlockSpec((1, 1, BN), lambda i: (i, 0, 0)),
        ],
        out_specs=[
            pl.BlockSpec((G, HP), lambda i: (0, 0)),
            pl.BlockSpec((G, 1), lambda i: (0, 0)),
        ],
        out_shape=[
            jax.ShapeDtypeStruct((G, HP), _f32),
            jax.ShapeDtypeStruct((G, 1), _f32),
        ],
    )(h, brow)


def _ro_context_body(h_ref, bcol_ref, brow_ref, gh_ref, roW_ref, rob_ref,
                     ctx_ref):
    i = pl.program_id(0)

    @pl.when(i == 0)
    def _init():
        ctx_ref[...] = jnp.zeros_like(ctx_ref)

    h = h_ref[...]
    Bm = jnp.where(bcol_ref[...] == lax.broadcasted_iota(jnp.int32, (BN, G), 1),
                   1.0, 0.0)
    BmT = jnp.where(brow_ref[0] == lax.broadcasted_iota(jnp.int32, (G, BN), 0),
                    1.0, 0.0)
    ctx = jnp.dot(Bm, gh_ref[...], preferred_element_type=_f32, precision=_HI)
    ap = _sigmoid(jnp.dot(h * ctx, roW_ref[...],
                          preferred_element_type=_f32, precision=_HI) + rob_ref[...])
    ctx_ref[...] += jnp.dot(BmT, ap * h,
                            preferred_element_type=_f32, precision=_HI)


def _ro_context(h, bcol, brow, gh, roW, rob):
    nb = N // BN
    w = lambda shape: pl.BlockSpec(shape, lambda i: (0, 0))
    return pl.pallas_call(
        _ro_context_body,
        grid=(nb,),
        in_specs=[
            pl.BlockSpec((BN, HP), lambda i: (i, 0)),
            pl.BlockSpec((BN, 1), lambda i: (i, 0)),
            pl.BlockSpec((1, 1, BN), lambda i: (i, 0, 0)),
            w((G, HP)), w((HP, 1)), w((1, 1)),
        ],
        out_specs=pl.BlockSpec((G, HP), lambda i: (0, 0)),
        out_shape=jax.ShapeDtypeStruct((G, HP), _f32),
    )(h, bcol, brow, gh, roW, rob)


def _gru_body(ctxt_ref, gh_ref,
              WihR_ref, WihZ_ref, WihN_ref, bihR_ref, bihZ_ref, bihN_ref,
              WhhR_ref, WhhZ_ref, WhhN_ref, bhhR_ref, bhhZ_ref, bhhN_ref,
              outW_ref, outb_ref, gh_out_ref, out_ref):
    gh = gh_ref[...]
    context = ctxt_ref[...]
    dot = lambda a, b: jnp.dot(a, b, preferred_element_type=_f32,
                               precision=_HI)
    i_r = dot(context, WihR_ref[...]) + bihR_ref[...]
    i_z = dot(context, WihZ_ref[...]) + bihZ_ref[...]
    i_n = dot(context, WihN_ref[...]) + bihN_ref[...]
    h_r = dot(gh, WhhR_ref[...]) + bhhR_ref[...]
    h_z = dot(gh, WhhZ_ref[...]) + bhhZ_ref[...]
    h_n = dot(gh, WhhN_ref[...]) + bhhN_ref[...]
    r = _sigmoid(i_r + h_r)
    z = _sigmoid(i_z + h_z)
    n = jnp.tanh(i_n + r * h_n)
    gh2 = (1.0 - z) * n + z * gh
    gh_out_ref[...] = gh2
    out_ref[...] = dot(gh2, outW_ref[...]) + outb_ref[...]


def _gru_step(context, gh, Wih, bih, Whh, bhh, outW, outb):
    return pl.pallas_call(
        _gru_body,
        out_shape=[
            jax.ShapeDtypeStruct((G, HP), _f32),
            jax.ShapeDtypeStruct((G, 1), _f32),
        ],
    )(context, gh, *Wih, *bih, *Whh, *bhh, outW, outb)


def _div_body(ghs_ref, cnt_ref, gh_ref):
    gh_ref[...] = ghs_ref[...] / jnp.maximum(cnt_ref[...], 1.0)


def _seg_div(ghs, cnt):
    return pl.pallas_call(
        _div_body,
        out_shape=jax.ShapeDtypeStruct((G, HP), _f32),
    )(ghs, cnt)


def _readout(h, bcol, brow, roW, rob, Wih, bih, Whh, bhh, outW, outb):
    ghs, cnt = _seg_mean(h, brow)
    gh = _seg_div(ghs, cnt)
    for _ in range(2):
        context = _ro_context(h, bcol, brow, gh, roW, rob)
        gh, out = _gru_step(context, gh, Wih, bih, Whh, bhh, outW, outb)
    return out


def _padc(w):
    return jnp.pad(w, ((0, 0), (0, HP - w.shape[1])))


def _padrc(w):
    return jnp.pad(w, ((0, HP - w.shape[0]), (0, HP - w.shape[1])))


def _padr(w):
    return jnp.pad(w, ((0, HP - w.shape[0]), (0, 0)))


def _padv(v):
    return jnp.pad(v, (0, HP - v.shape[0])).reshape(1, HP)


def kernel(node_features, edge_features, edge_index, batch,
           np_W, np_b, ep_W, ep_b,
           l0_attn_W, l0_attn_b, l0_msg_W, l0_msg_b, l0_ln_g, l0_ln_b,
           l1_attn_W, l1_attn_b, l1_msg_W, l1_msg_b, l1_ln_g, l1_ln_b,
           ro_W, ro_b, gru_Wih, gru_bih, gru_Whh, gru_bhh, out_W, out_b):
    npW = _padc(np_W); npb = _padv(np_b)
    epW = _padc(ep_W); epb = _padv(ep_b)
    aW0s = _padr(l0_attn_W[:H]); aW0d = _padr(l0_attn_W[H:2 * H])
    aW0e = _padr(l0_attn_W[2 * H:]); ab0 = l0_attn_b.reshape(1, 1)
    aW1s = _padr(l1_attn_W[:H]); aW1d = _padr(l1_attn_W[H:2 * H])
    aW1e = _padr(l1_attn_W[2 * H:]); ab1 = l1_attn_b.reshape(1, 1)
    mW0h = _padrc(l0_msg_W[:H]); mW0e = _padrc(l0_msg_W[H:])
    mW1h = _padrc(l1_msg_W[:H]); mW1e = _padrc(l1_msg_W[H:])
    mW0ha, mW0hb = mW0h[:, :HH], mW0h[:, HH:]
    mW0ea, mW0eb = mW0e[:, :HH], mW0e[:, HH:]
    mW1ha, mW1hb = mW1h[:, :HH], mW1h[:, HH:]
    mW1ea, mW1eb = mW1e[:, :HH], mW1e[:, HH:]
    mb0 = _padv(l0_msg_b); mb1 = _padv(l1_msg_b)
    g0 = _padv(l0_ln_g); b0 = _padv(l0_ln_b)
    g1 = _padv(l1_ln_g); b1 = _padv(l1_ln_b)
    roW = _padr(ro_W); rob = ro_b.reshape(1, 1)
    Wih = [_padrc(gru_Wih[:, i * H:(i + 1) * H]) for i in range(3)]
    bih = [_padv(gru_bih[i * H:(i + 1) * H]) for i in range(3)]
    Whh = [_padrc(gru_Whh[:, i * H:(i + 1) * H]) for i in range(3)]
    bhh = [_padv(gru_bhh[i * H:(i + 1) * H]) for i in range(3)]
    outW = _padr(out_W); outb = out_b.reshape(1, 1)

    src = jnp.pad(edge_index[0], (0, EPAD - E))
    dst = jnp.pad(edge_index[1], (0, EPAD - E))
    epad_blk = -(-EPAD // BE) * BE
    efp = jnp.pad(edge_features, ((0, epad_blk - E), (0, 0)))
    bcol = batch.reshape(N, 1)
    brow = batch.reshape(N // BN, 1, BN)

    Pa0, Pb0, D0, S0 = _node_prep(node_features, npW, npb, mW0ha, mW0hb,
                                  aW0s, aW0d)
    Qa0, Qb0, ae0, Qa1, Qb1, ae1 = _edge_prep(
        efp, epW, epb, mW0ea, mW0eb, mb0, aW0e, ab0,
        mW1ea, mW1eb, mb1, aW1e, ab1)

    sc = _get_sc_edge()
    scl = _get_sc_logit()
    al0 = _tc_sigmoid(scl(S0, D0, ae0.reshape(EPAD), src, dst))
    parts0 = sc(Pa0, Pb0, Qa0, Qb0, al0, src, dst)
    _, Pa1, Pb1, D1, S1 = _combine(parts0, g0, b0, mW1ha, mW1hb, aW1s, aW1d)

    al1 = _tc_sigmoid(scl(S1, D1, ae1.reshape(EPAD), src, dst))
    parts1 = sc(Pa1, Pb1, Qa1, Qb1, al1, src, dst)
    h2, _, _, _, _ = _combine(parts1, g1, b1, mW1ha, mW1hb, aW1s, aW1d)

    return _readout(h2, bcol, brow, roW, rob, Wih, bih, Whh, bhh, outW, outb)
